# Initial kernel scaffold; baseline (speedup 1.0000x reference)
import jax
import jax.numpy as jnp
from jax.experimental import pallas as pl


def kernel(x, W_gcn, b_gcn, W_out, b_out, edge_index, cluster_ids):
    raise NotImplementedError("write your pallas kernel here")


# speedup vs baseline: 8.0757x; 8.0757x over previous
import functools

import jax
import jax.numpy as jnp
from jax import lax
from jax.experimental import pallas as pl
from jax.experimental.pallas import tpu as pltpu
from jax.experimental.pallas import tpu_sc as plsc

N = 10000
F = 256
E = 320000
NCLUS = 5000
HALF = 128
CH = 128

NP = 10240
EP = 327680
NCP = 5120

_MESH = plsc.VectorSubcoreMesh(core_axis_name="c", subcore_axis_name="s")


def _zero_rows(ref, nrows, ncols):
    z = jnp.zeros((16,), jnp.float32)

    def body(r, carry):
        for k in range(ncols // 16):
            ref[r, pl.ds(k * 16, 16)] = z
        return carry

    lax.fori_loop(0, nrows, body, 0)


@functools.partial(
    pl.kernel,
    out_type=(
        jax.ShapeDtypeStruct((2 * NP, 16), jnp.float32),
        jax.ShapeDtypeStruct((NCP, 16), jnp.float32),
    ),
    mesh=_MESH,
    scratch_types=[
        pltpu.VMEM((CH,), jnp.int32),
        pltpu.VMEM((CH, 16), jnp.float32),
        pltpu.VMEM((CH, 16), jnp.float32),
        pltpu.VMEM_SHARED((NP, 16), jnp.float32),
        pltpu.VMEM_SHARED((NCP, 16), jnp.float32),
    ],
)
def _sc_degree_counts(dst_hbm, cl_hbm, deg_out, cnt_out,
                      idx_b, one_b, zero_b, deg_acc, cnt_acc):
    c = lax.axis_index("c")
    s = lax.axis_index("s")
    iota = lax.iota(jnp.int32, 16)
    oh = jnp.where(iota == 0, 1.0, 0.0).astype(jnp.float32)
    z = jnp.zeros((16,), jnp.float32)

    def fill(r, carry):
        one_b[r, pl.ds(0, 16)] = oh
        zero_b[r, pl.ds(0, 16)] = z
        return carry

    lax.fori_loop(0, CH, fill, 0)

    for k in range(5):
        pltpu.sync_copy(zero_b, deg_acc.at[pl.ds(s * 640 + k * CH, CH)])
    for k in range(2):
        pltpu.sync_copy(zero_b, cnt_acc.at[pl.ds(s * 320 + k * CH, CH)])
    pltpu.sync_copy(zero_b.at[pl.ds(0, 64)], cnt_acc.at[pl.ds(s * 320 + 256, 64)])
    plsc.subcore_barrier()

    def dbody(j, carry):
        base = ((c * 16 + s) * 80 + j) * CH
        pltpu.sync_copy(dst_hbm.at[pl.ds(base, CH)], idx_b)
        pltpu.sync_copy(one_b, deg_acc.at[idx_b], add=True)
        return carry

    lax.fori_loop(0, 80, dbody, 0)

    @pl.when(c == 0)
    def _():
        def cbody(j, carry):
            base = (s * 5 + j) * CH
            pltpu.sync_copy(cl_hbm.at[pl.ds(base, CH)], idx_b)
            pltpu.sync_copy(one_b, cnt_acc.at[idx_b], add=True)
            return carry

        lax.fori_loop(0, 5, cbody, 0)

    plsc.subcore_barrier()
    pltpu.sync_copy(deg_acc.at[pl.ds(s * 640, 640)],
                    deg_out.at[pl.ds(c * NP + s * 640, 640)])

    @pl.when(c == 0)
    def _():
        pltpu.sync_copy(cnt_acc.at[pl.ds(s * 320, 320)],
                        cnt_out.at[pl.ds(s * 320, 320)])


@functools.partial(
    pl.kernel,
    out_type=jax.ShapeDtypeStruct((2 * NP, HALF), jnp.float32),
    mesh=_MESH,
    scratch_types=[
        pltpu.VMEM((CH,), jnp.int32),
        pltpu.VMEM((CH,), jnp.int32),
        pltpu.VMEM((CH, HALF), jnp.float32),
        pltpu.VMEM_SHARED((NP, HALF), jnp.float32),
        pltpu.SemaphoreType.DMA,
    ],
)
def _sc_edge_agg(y_hbm, src_hbm, dst_hbm, out_hbm,
                 src_b, dst_b, rows_b, acc, sem):
    c = lax.axis_index("c")
    s = lax.axis_index("s")
    _zero_rows(rows_b, CH, HALF)
    for k in range(5):
        pltpu.sync_copy(rows_b, acc.at[pl.ds(s * 640 + k * CH, CH)])
    plsc.subcore_barrier()

    off = c * NP

    def ebody(j, carry):
        base = (s * 160 + j) * CH
        pltpu.sync_copy(src_hbm.at[pl.ds(base, CH)], src_b)
        pltpu.sync_copy(dst_hbm.at[pl.ds(base, CH)], dst_b)
        for k in range(CH // 16):
            src_b[pl.ds(k * 16, 16)] = src_b[pl.ds(k * 16, 16)] + off
        pltpu.async_copy(y_hbm.at[src_b], rows_b, sem).wait()
        pltpu.sync_copy(rows_b, acc.at[dst_b], add=True)
        return carry

    lax.fori_loop(0, 160, ebody, 0)
    plsc.subcore_barrier()
    pltpu.sync_copy(acc.at[pl.ds(s * 640, 640)],
                    out_hbm.at[pl.ds(c * NP + s * 640, 640)])


@functools.partial(
    pl.kernel,
    out_type=jax.ShapeDtypeStruct((2 * NCP, HALF), jnp.float32),
    mesh=_MESH,
    scratch_types=[
        pltpu.VMEM((CH,), jnp.int32),
        pltpu.VMEM((CH, HALF), jnp.float32),
        pltpu.VMEM_SHARED((NCP, HALF), jnp.float32),
    ],
)
def _sc_pool(h_hbm, cl_hbm, out_hbm, idx_b, rows_b, acc):
    c = lax.axis_index("c")
    s = lax.axis_index("s")
    _zero_rows(rows_b, CH, HALF)
    for k in range(2):
        pltpu.sync_copy(rows_b, acc.at[pl.ds(s * 320 + k * CH, CH)])
    pltpu.sync_copy(rows_b.at[pl.ds(0, 64)], acc.at[pl.ds(s * 320 + 256, 64)])
    plsc.subcore_barrier()

    def pbody(j, carry):
        nbase = s * 640 + j * CH
        pltpu.sync_copy(cl_hbm.at[pl.ds(nbase, CH)], idx_b)
        pltpu.sync_copy(h_hbm.at[pl.ds(c * NP + nbase, CH)], rows_b)
        pltpu.sync_copy(rows_b, acc.at[idx_b], add=True)
        return carry

    lax.fori_loop(0, 5, pbody, 0)
    plsc.subcore_barrier()
    pltpu.sync_copy(acc.at[pl.ds(s * 320, 320)],
                    out_hbm.at[pl.ds(c * NCP + s * 320, 320)])


def _tc1_body(x_ref, w_ref, degp_ref, y_ref):
    xw = jnp.dot(x_ref[...], w_ref[...], preferred_element_type=jnp.float32)
    d = degp_ref[...]
    deg = d[0, :, 0:1] + d[1, :, 0:1] + 1.0
    y = xw * lax.rsqrt(deg)
    y_ref[0, :, :] = y[:, :HALF]
    y_ref[1, :, :] = y[:, HALF:]


def _tc2_body(y_ref, a_ref, degp_ref, b_ref, h_ref):
    d = degp_ref[...]
    dinv = lax.rsqrt(d[0, :, 0:1] + d[1, :, 0:1] + 1.0)
    for half in range(2):
        h_ref[half, :, :] = jnp.tanh(
            (y_ref[half, :, :] + a_ref[half, :, :]) * dinv + b_ref[half, :, :])


def _tc3_body(p_ref, cnt_ref, w_ref, b_ref, o_ref):
    inv = 1.0 / jnp.maximum(cnt_ref[:, 0:1], 1.0)
    p0 = p_ref[0, :, :] * inv
    p1 = p_ref[1, :, :] * inv
    o_ref[...] = (jnp.dot(p0, w_ref[:HALF, :], preferred_element_type=jnp.float32)
                  + jnp.dot(p1, w_ref[HALF:, :], preferred_element_type=jnp.float32)
                  + b_ref[...])


_BM = 256


def _tc1(x_pad, w, deg_parts):
    return pl.pallas_call(
        _tc1_body,
        grid=(NP // _BM,),
        in_specs=[pl.BlockSpec((_BM, F), lambda i: (i, 0)),
                  pl.BlockSpec((F, F), lambda i: (0, 0)),
                  pl.BlockSpec((2, _BM, 16), lambda i: (0, i, 0))],
        out_specs=pl.BlockSpec((2, _BM, HALF), lambda i: (0, i, 0)),
        out_shape=jax.ShapeDtypeStruct((2, NP, HALF), jnp.float32),
    )(x_pad, w, deg_parts)


def _tc2(y, agg, deg_parts, b2):
    return pl.pallas_call(
        _tc2_body,
        grid=(NP // _BM,),
        in_specs=[pl.BlockSpec((2, _BM, HALF), lambda i: (0, i, 0)),
                  pl.BlockSpec((2, _BM, HALF), lambda i: (0, i, 0)),
                  pl.BlockSpec((2, _BM, 16), lambda i: (0, i, 0)),
                  pl.BlockSpec((2, 1, HALF), lambda i: (0, 0, 0))],
        out_specs=pl.BlockSpec((2, _BM, HALF), lambda i: (0, i, 0)),
        out_shape=jax.ShapeDtypeStruct((2, NP, HALF), jnp.float32),
    )(y, agg, deg_parts, b2)


def _tc3(pooled, cnt, w, b2):
    bm = 512
    return pl.pallas_call(
        _tc3_body,
        grid=(NCP // bm,),
        in_specs=[pl.BlockSpec((2, bm, HALF), lambda i: (0, i, 0)),
                  pl.BlockSpec((bm, 16), lambda i: (i, 0)),
                  pl.BlockSpec((F, F), lambda i: (0, 0)),
                  pl.BlockSpec((1, F), lambda i: (0, 0))],
        out_specs=pl.BlockSpec((bm, F), lambda i: (i, 0)),
        out_shape=jax.ShapeDtypeStruct((NCP, F), jnp.float32),
    )(pooled, cnt, w, b2)


def kernel(x, W_gcn, b_gcn, W_out, b_out, edge_index, cluster_ids):
    src = jnp.concatenate(
        [edge_index[0], jnp.full((EP - E,), N, jnp.int32)])
    dst = jnp.concatenate(
        [edge_index[1], jnp.full((EP - E,), N, jnp.int32)])
    cl = jnp.concatenate(
        [cluster_ids, jnp.full((NP - N,), NCLUS, jnp.int32)])
    x_pad = jnp.pad(x, ((0, NP - N), (0, 0)))

    degf, cnt = _sc_degree_counts(dst, cl)
    deg_parts = degf.reshape(2, NP, 16)
    y = _tc1(x_pad, W_gcn, deg_parts)
    agg = _sc_edge_agg(y.reshape(2 * NP, HALF), src, dst)
    h = _tc2(y, agg.reshape(2, NP, HALF), deg_parts, b_gcn.reshape(2, 1, HALF))
    pooled = _sc_pool(h.reshape(2 * NP, HALF), cl)
    outp = _tc3(pooled.reshape(2, NCP, HALF), cnt, W_out, b_out.reshape(1, F))
    return outp[:NCLUS]


# speedup vs baseline: 8.8865x; 1.1004x over previous
import functools

import jax
import jax.numpy as jnp
from jax import lax
from jax.experimental import pallas as pl
from jax.experimental.pallas import tpu as pltpu
from jax.experimental.pallas import tpu_sc as plsc

N = 10000
F = 256
E = 320000
NCLUS = 5000
HALF = 128
CH = 128

NP = 10240
EP = 327680
NCP = 5120

_MESH = plsc.VectorSubcoreMesh(core_axis_name="c", subcore_axis_name="s")


def _zero_rows(ref, nrows, ncols):
    z = jnp.zeros((16,), jnp.float32)

    def body(r, carry):
        for k in range(ncols // 16):
            ref[r, pl.ds(k * 16, 16)] = z
        return carry

    lax.fori_loop(0, nrows, body, 0)


@functools.partial(
    pl.kernel,
    out_type=(
        jax.ShapeDtypeStruct((2 * NP, 16), jnp.float32),
        jax.ShapeDtypeStruct((NCP, 16), jnp.float32),
    ),
    mesh=_MESH,
    scratch_types=[
        pltpu.VMEM((CH,), jnp.int32),
        pltpu.VMEM((CH, 16), jnp.float32),
        pltpu.VMEM((CH, 16), jnp.float32),
        pltpu.VMEM_SHARED((NP, 16), jnp.float32),
        pltpu.VMEM_SHARED((NCP, 16), jnp.float32),
    ],
)
def _sc_degree_counts(dst_hbm, cl_hbm, deg_out, cnt_out,
                      idx_b, one_b, zero_b, deg_acc, cnt_acc):
    c = lax.axis_index("c")
    s = lax.axis_index("s")
    iota = lax.iota(jnp.int32, 16)
    oh = jnp.where(iota == 0, 1.0, 0.0).astype(jnp.float32)
    z = jnp.zeros((16,), jnp.float32)

    def fill(r, carry):
        one_b[r, pl.ds(0, 16)] = oh
        zero_b[r, pl.ds(0, 16)] = z
        return carry

    lax.fori_loop(0, CH, fill, 0)

    for k in range(5):
        pltpu.sync_copy(zero_b, deg_acc.at[pl.ds(s * 640 + k * CH, CH)])
    for k in range(2):
        pltpu.sync_copy(zero_b, cnt_acc.at[pl.ds(s * 320 + k * CH, CH)])
    pltpu.sync_copy(zero_b.at[pl.ds(0, 64)], cnt_acc.at[pl.ds(s * 320 + 256, 64)])
    plsc.subcore_barrier()

    def dbody(j, carry):
        base = ((c * 16 + s) * 80 + j) * CH
        pltpu.sync_copy(dst_hbm.at[pl.ds(base, CH)], idx_b)
        pltpu.sync_copy(one_b, deg_acc.at[idx_b], add=True)
        return carry

    lax.fori_loop(0, 80, dbody, 0)

    @pl.when(c == 0)
    def _():
        def cbody(j, carry):
            base = (s * 5 + j) * CH
            pltpu.sync_copy(cl_hbm.at[pl.ds(base, CH)], idx_b)
            pltpu.sync_copy(one_b, cnt_acc.at[idx_b], add=True)
            return carry

        lax.fori_loop(0, 5, cbody, 0)

    plsc.subcore_barrier()
    pltpu.sync_copy(deg_acc.at[pl.ds(s * 640, 640)],
                    deg_out.at[pl.ds(c * NP + s * 640, 640)])

    @pl.when(c == 0)
    def _():
        pltpu.sync_copy(cnt_acc.at[pl.ds(s * 320, 320)],
                        cnt_out.at[pl.ds(s * 320, 320)])


@functools.partial(
    pl.kernel,
    out_type=jax.ShapeDtypeStruct((2 * NP, HALF), jnp.float32),
    mesh=_MESH,
    scratch_types=[
        pltpu.VMEM((CH,), jnp.int32),
        pltpu.VMEM((CH,), jnp.int32),
        pltpu.VMEM((CH,), jnp.int32),
        pltpu.VMEM((CH,), jnp.int32),
        pltpu.VMEM((CH, HALF), jnp.float32),
        pltpu.VMEM((CH, HALF), jnp.float32),
        pltpu.VMEM_SHARED((NP, HALF), jnp.float32),
        pltpu.SemaphoreType.DMA,
        pltpu.SemaphoreType.DMA,
    ],
)
def _sc_edge_agg(y_hbm, src_hbm, dst_hbm, out_hbm,
                 src_b0, src_b1, dst_b0, dst_b1, rows_b0, rows_b1,
                 acc, sem0, sem1):
    c = lax.axis_index("c")
    s = lax.axis_index("s")
    srcs = (src_b0, src_b1)
    dsts = (dst_b0, dst_b1)
    rows = (rows_b0, rows_b1)
    sems = (sem0, sem1)
    _zero_rows(rows_b0, CH, HALF)
    for k in range(5):
        pltpu.sync_copy(rows_b0, acc.at[pl.ds(s * 640 + k * CH, CH)])
    plsc.subcore_barrier()

    off = c * NP

    def load_idx(j, p):
        base = (s * 160 + j) * CH
        pltpu.sync_copy(src_hbm.at[pl.ds(base, CH)], srcs[p])
        pltpu.sync_copy(dst_hbm.at[pl.ds(base, CH)], dsts[p])
        for k in range(CH // 16):
            srcs[p][pl.ds(k * 16, 16)] = srcs[p][pl.ds(k * 16, 16)] + off

    load_idx(0, 0)
    pltpu.async_copy(y_hbm.at[srcs[0]], rows[0], sems[0])

    def ebody(i, carry):
        for p in (0, 1):
            j = 2 * i + p
            q = 1 - p
            pltpu.make_async_copy(y_hbm.at[srcs[p]], rows[p], sems[p]).wait()

            @pl.when(j < 159)
            def _():
                load_idx(j + 1, q)
                pltpu.async_copy(y_hbm.at[srcs[q]], rows[q], sems[q])

            pltpu.sync_copy(rows[p], acc.at[dsts[p]], add=True)
        return carry

    lax.fori_loop(0, 80, ebody, 0)
    plsc.subcore_barrier()
    pltpu.sync_copy(acc.at[pl.ds(s * 640, 640)],
                    out_hbm.at[pl.ds(c * NP + s * 640, 640)])


@functools.partial(
    pl.kernel,
    out_type=jax.ShapeDtypeStruct((2 * NCP, HALF), jnp.float32),
    mesh=_MESH,
    scratch_types=[
        pltpu.VMEM((CH,), jnp.int32),
        pltpu.VMEM((CH, HALF), jnp.float32),
        pltpu.VMEM_SHARED((NCP, HALF), jnp.float32),
    ],
)
def _sc_pool(h_hbm, cl_hbm, out_hbm, idx_b, rows_b, acc):
    c = lax.axis_index("c")
    s = lax.axis_index("s")
    _zero_rows(rows_b, CH, HALF)
    for k in range(2):
        pltpu.sync_copy(rows_b, acc.at[pl.ds(s * 320 + k * CH, CH)])
    pltpu.sync_copy(rows_b.at[pl.ds(0, 64)], acc.at[pl.ds(s * 320 + 256, 64)])
    plsc.subcore_barrier()

    def pbody(j, carry):
        nbase = s * 640 + j * CH
        pltpu.sync_copy(cl_hbm.at[pl.ds(nbase, CH)], idx_b)
        pltpu.sync_copy(h_hbm.at[pl.ds(c * NP + nbase, CH)], rows_b)
        pltpu.sync_copy(rows_b, acc.at[idx_b], add=True)
        return carry

    lax.fori_loop(0, 5, pbody, 0)
    plsc.subcore_barrier()
    pltpu.sync_copy(acc.at[pl.ds(s * 320, 320)],
                    out_hbm.at[pl.ds(c * NCP + s * 320, 320)])


def _tc1_body(x_ref, w_ref, degp_ref, y_ref):
    xw = jnp.dot(x_ref[...], w_ref[...], preferred_element_type=jnp.float32)
    d = degp_ref[...]
    deg = d[0, :, 0:1] + d[1, :, 0:1] + 1.0
    y = xw * lax.rsqrt(deg)
    y_ref[0, :, :] = y[:, :HALF]
    y_ref[1, :, :] = y[:, HALF:]


def _tc2_body(y_ref, a_ref, degp_ref, b_ref, h_ref):
    d = degp_ref[...]
    dinv = lax.rsqrt(d[0, :, 0:1] + d[1, :, 0:1] + 1.0)
    for half in range(2):
        h_ref[half, :, :] = jnp.tanh(
            (y_ref[half, :, :] + a_ref[half, :, :]) * dinv + b_ref[half, :, :])


def _tc3_body(p_ref, cnt_ref, w_ref, b_ref, o_ref):
    inv = 1.0 / jnp.maximum(cnt_ref[:, 0:1], 1.0)
    p0 = p_ref[0, :, :] * inv
    p1 = p_ref[1, :, :] * inv
    o_ref[...] = (jnp.dot(p0, w_ref[:HALF, :], preferred_element_type=jnp.float32)
                  + jnp.dot(p1, w_ref[HALF:, :], preferred_element_type=jnp.float32)
                  + b_ref[...])


_BM = 256


def _tc1(x_pad, w, deg_parts):
    return pl.pallas_call(
        _tc1_body,
        grid=(NP // _BM,),
        in_specs=[pl.BlockSpec((_BM, F), lambda i: (i, 0)),
                  pl.BlockSpec((F, F), lambda i: (0, 0)),
                  pl.BlockSpec((2, _BM, 16), lambda i: (0, i, 0))],
        out_specs=pl.BlockSpec((2, _BM, HALF), lambda i: (0, i, 0)),
        out_shape=jax.ShapeDtypeStruct((2, NP, HALF), jnp.float32),
    )(x_pad, w, deg_parts)


def _tc2(y, agg, deg_parts, b2):
    return pl.pallas_call(
        _tc2_body,
        grid=(NP // _BM,),
        in_specs=[pl.BlockSpec((2, _BM, HALF), lambda i: (0, i, 0)),
                  pl.BlockSpec((2, _BM, HALF), lambda i: (0, i, 0)),
                  pl.BlockSpec((2, _BM, 16), lambda i: (0, i, 0)),
                  pl.BlockSpec((2, 1, HALF), lambda i: (0, 0, 0))],
        out_specs=pl.BlockSpec((2, _BM, HALF), lambda i: (0, i, 0)),
        out_shape=jax.ShapeDtypeStruct((2, NP, HALF), jnp.float32),
    )(y, agg, deg_parts, b2)


def _tc3(pooled, cnt, w, b2):
    bm = 512
    return pl.pallas_call(
        _tc3_body,
        grid=(NCP // bm,),
        in_specs=[pl.BlockSpec((2, bm, HALF), lambda i: (0, i, 0)),
                  pl.BlockSpec((bm, 16), lambda i: (i, 0)),
                  pl.BlockSpec((F, F), lambda i: (0, 0)),
                  pl.BlockSpec((1, F), lambda i: (0, 0))],
        out_specs=pl.BlockSpec((bm, F), lambda i: (i, 0)),
        out_shape=jax.ShapeDtypeStruct((NCP, F), jnp.float32),
    )(pooled, cnt, w, b2)


def kernel(x, W_gcn, b_gcn, W_out, b_out, edge_index, cluster_ids):
    src = jnp.concatenate(
        [edge_index[0], jnp.full((EP - E,), N, jnp.int32)])
    dst = jnp.concatenate(
        [edge_index[1], jnp.full((EP - E,), N, jnp.int32)])
    cl = jnp.concatenate(
        [cluster_ids, jnp.full((NP - N,), NCLUS, jnp.int32)])
    x_pad = jnp.pad(x, ((0, NP - N), (0, 0)))

    degf, cnt = _sc_degree_counts(dst, cl)
    deg_parts = degf.reshape(2, NP, 16)
    y = _tc1(x_pad, W_gcn, deg_parts)
    agg = _sc_edge_agg(y.reshape(2 * NP, HALF), src, dst)
    h = _tc2(y, agg.reshape(2, NP, HALF), deg_parts, b_gcn.reshape(2, 1, HALF))
    pooled = _sc_pool(h.reshape(2 * NP, HALF), cl)
    outp = _tc3(pooled.reshape(2, NCP, HALF), cnt, W_out, b_out.reshape(1, F))
    return outp[:NCLUS]


# speedup vs baseline: 9.6920x; 1.0906x over previous
import functools

import jax
import jax.numpy as jnp
from jax import lax
from jax.experimental import pallas as pl
from jax.experimental.pallas import tpu as pltpu
from jax.experimental.pallas import tpu_sc as plsc

N = 10000
F = 256
E = 320000
NCLUS = 5000
HALF = 128
CH = 128

NP = 10240
EP = 327680
NCP = 5120

_MESH = plsc.VectorSubcoreMesh(core_axis_name="c", subcore_axis_name="s")


def _zero_rows(ref, nrows, ncols):
    z = jnp.zeros((16,), jnp.float32)

    def body(r, carry):
        for k in range(ncols // 16):
            ref[r, pl.ds(k * 16, 16)] = z
        return carry

    lax.fori_loop(0, nrows, body, 0)


@functools.partial(
    pl.kernel,
    out_type=(
        jax.ShapeDtypeStruct((2 * NP, 16), jnp.float32),
        jax.ShapeDtypeStruct((NCP, 16), jnp.float32),
    ),
    mesh=_MESH,
    scratch_types=[
        pltpu.VMEM((CH,), jnp.int32),
        pltpu.VMEM((CH, 16), jnp.float32),
        pltpu.VMEM((CH, 16), jnp.float32),
        pltpu.VMEM_SHARED((NP, 16), jnp.float32),
        pltpu.VMEM_SHARED((NCP, 16), jnp.float32),
    ],
)
def _sc_degree_counts(dst_hbm, cl_hbm, deg_out, cnt_out,
                      idx_b, one_b, zero_b, deg_acc, cnt_acc):
    c = lax.axis_index("c")
    s = lax.axis_index("s")
    iota = lax.iota(jnp.int32, 16)
    oh = jnp.where(iota == 0, 1.0, 0.0).astype(jnp.float32)
    z = jnp.zeros((16,), jnp.float32)

    def fill(r, carry):
        one_b[r, pl.ds(0, 16)] = oh
        zero_b[r, pl.ds(0, 16)] = z
        return carry

    lax.fori_loop(0, CH, fill, 0)

    for k in range(5):
        pltpu.sync_copy(zero_b, deg_acc.at[pl.ds(s * 640 + k * CH, CH)])
    for k in range(2):
        pltpu.sync_copy(zero_b, cnt_acc.at[pl.ds(s * 320 + k * CH, CH)])
    pltpu.sync_copy(zero_b.at[pl.ds(0, 64)], cnt_acc.at[pl.ds(s * 320 + 256, 64)])
    plsc.subcore_barrier()

    def dbody(j, carry):
        base = ((c * 16 + s) * 80 + j) * CH
        pltpu.sync_copy(dst_hbm.at[pl.ds(base, CH)], idx_b)
        pltpu.sync_copy(one_b, deg_acc.at[idx_b], add=True)
        return carry

    lax.fori_loop(0, 80, dbody, 0)

    @pl.when(c == 0)
    def _():
        def cbody(j, carry):
            base = (s * 5 + j) * CH
            pltpu.sync_copy(cl_hbm.at[pl.ds(base, CH)], idx_b)
            pltpu.sync_copy(one_b, cnt_acc.at[idx_b], add=True)
            return carry

        lax.fori_loop(0, 5, cbody, 0)

    plsc.subcore_barrier()
    pltpu.sync_copy(deg_acc.at[pl.ds(s * 640, 640)],
                    deg_out.at[pl.ds(c * NP + s * 640, 640)])

    @pl.when(c == 0)
    def _():
        pltpu.sync_copy(cnt_acc.at[pl.ds(s * 320, 320)],
                        cnt_out.at[pl.ds(s * 320, 320)])


_NBUF = 5
_ECH = 64
_ECHUNKS = 20480 // _ECH


@functools.partial(
    pl.kernel,
    out_type=jax.ShapeDtypeStruct((2 * NP, HALF), jnp.float32),
    mesh=_MESH,
    scratch_types=(
        [pltpu.VMEM((_ECH,), jnp.int32) for _ in range(_NBUF)]
        + [pltpu.VMEM((_ECH,), jnp.int32) for _ in range(_NBUF)]
        + [pltpu.VMEM((_ECH, HALF), jnp.float32) for _ in range(_NBUF)]
        + [pltpu.VMEM_SHARED((NP, HALF), jnp.float32)]
        + [pltpu.SemaphoreType.DMA for _ in range(2 * _NBUF)]
    ),
)
def _sc_edge_agg(y_hbm, src2_hbm, dst_hbm, out_hbm, *refs):
    srcs = refs[0:_NBUF]
    dsts = refs[_NBUF:2 * _NBUF]
    rows = refs[2 * _NBUF:3 * _NBUF]
    acc = refs[3 * _NBUF]
    gsem = refs[3 * _NBUF + 1:3 * _NBUF + 1 + _NBUF]
    ssem = refs[3 * _NBUF + 1 + _NBUF:]
    c = lax.axis_index("c")
    s = lax.axis_index("s")
    _zero_rows(rows[0], _ECH, HALF)
    for k in range(640 // _ECH):
        pltpu.sync_copy(rows[0], acc.at[pl.ds(s * 640 + k * _ECH, _ECH)])
    plsc.subcore_barrier()

    def load_idx(j, p):
        base = (s * _ECHUNKS + j) * _ECH
        pltpu.sync_copy(src2_hbm.at[pl.ds(c * EP + base, _ECH)], srcs[p])
        pltpu.sync_copy(dst_hbm.at[pl.ds(base, _ECH)], dsts[p])

    load_idx(0, 0)
    pltpu.async_copy(y_hbm.at[srcs[0]], rows[0], gsem[0])
    load_idx(1, 1)
    pltpu.async_copy(y_hbm.at[srcs[1]], rows[1], gsem[1])

    def ebody(i, carry):
        for p in range(_NBUF):
            j = _NBUF * i + p
            n = (p + 2) % _NBUF
            pltpu.make_async_copy(y_hbm.at[srcs[p]], rows[p], gsem[p]).wait()

            @pl.when(j < _ECHUNKS - 2)
            def _():
                @pl.when(j >= _NBUF - 2)
                def _():
                    pltpu.make_async_copy(rows[n], acc.at[dsts[n]],
                                          ssem[n]).wait()
                load_idx(j + 2, n)
                pltpu.async_copy(y_hbm.at[srcs[n]], rows[n], gsem[n])

            pltpu.async_copy(rows[p], acc.at[dsts[p]], ssem[p], add=True)
        return carry

    lax.fori_loop(0, _ECHUNKS // _NBUF, ebody, 0)
    for p in range(_NBUF):
        pltpu.make_async_copy(rows[p], acc.at[dsts[p]], ssem[p]).wait()
    plsc.subcore_barrier()
    pltpu.sync_copy(acc.at[pl.ds(s * 640, 640)],
                    out_hbm.at[pl.ds(c * NP + s * 640, 640)])


@functools.partial(
    pl.kernel,
    out_type=jax.ShapeDtypeStruct((2 * NCP, HALF), jnp.float32),
    mesh=_MESH,
    scratch_types=[
        pltpu.VMEM((CH,), jnp.int32),
        pltpu.VMEM((CH, HALF), jnp.float32),
        pltpu.VMEM_SHARED((NCP, HALF), jnp.float32),
    ],
)
def _sc_pool(h_hbm, cl_hbm, out_hbm, idx_b, rows_b, acc):
    c = lax.axis_index("c")
    s = lax.axis_index("s")
    _zero_rows(rows_b, CH, HALF)
    for k in range(2):
        pltpu.sync_copy(rows_b, acc.at[pl.ds(s * 320 + k * CH, CH)])
    pltpu.sync_copy(rows_b.at[pl.ds(0, 64)], acc.at[pl.ds(s * 320 + 256, 64)])
    plsc.subcore_barrier()

    def pbody(j, carry):
        nbase = s * 640 + j * CH
        pltpu.sync_copy(cl_hbm.at[pl.ds(nbase, CH)], idx_b)
        pltpu.sync_copy(h_hbm.at[pl.ds(c * NP + nbase, CH)], rows_b)
        pltpu.sync_copy(rows_b, acc.at[idx_b], add=True)
        return carry

    lax.fori_loop(0, 5, pbody, 0)
    plsc.subcore_barrier()
    pltpu.sync_copy(acc.at[pl.ds(s * 320, 320)],
                    out_hbm.at[pl.ds(c * NCP + s * 320, 320)])


def _tc1_body(x_ref, w_ref, degp_ref, y_ref):
    xw = jnp.dot(x_ref[...], w_ref[...], preferred_element_type=jnp.float32)
    d = degp_ref[...]
    deg = d[0, :, 0:1] + d[1, :, 0:1] + 1.0
    y = xw * lax.rsqrt(deg)
    y_ref[0, :, :] = y[:, :HALF]
    y_ref[1, :, :] = y[:, HALF:]


def _tc2_body(y_ref, a_ref, degp_ref, b_ref, h_ref):
    d = degp_ref[...]
    dinv = lax.rsqrt(d[0, :, 0:1] + d[1, :, 0:1] + 1.0)
    for half in range(2):
        h_ref[half, :, :] = jnp.tanh(
            (y_ref[half, :, :] + a_ref[half, :, :]) * dinv + b_ref[half, :, :])


def _tc3_body(p_ref, cnt_ref, w_ref, b_ref, o_ref):
    inv = 1.0 / jnp.maximum(cnt_ref[:, 0:1], 1.0)
    p0 = p_ref[0, :, :] * inv
    p1 = p_ref[1, :, :] * inv
    o_ref[...] = (jnp.dot(p0, w_ref[:HALF, :], preferred_element_type=jnp.float32)
                  + jnp.dot(p1, w_ref[HALF:, :], preferred_element_type=jnp.float32)
                  + b_ref[...])


_BM = 256


def _tc1(x_pad, w, deg_parts):
    return pl.pallas_call(
        _tc1_body,
        grid=(NP // _BM,),
        in_specs=[pl.BlockSpec((_BM, F), lambda i: (i, 0)),
                  pl.BlockSpec((F, F), lambda i: (0, 0)),
                  pl.BlockSpec((2, _BM, 16), lambda i: (0, i, 0))],
        out_specs=pl.BlockSpec((2, _BM, HALF), lambda i: (0, i, 0)),
        out_shape=jax.ShapeDtypeStruct((2, NP, HALF), jnp.float32),
    )(x_pad, w, deg_parts)


def _tc2(y, agg, deg_parts, b2):
    return pl.pallas_call(
        _tc2_body,
        grid=(NP // _BM,),
        in_specs=[pl.BlockSpec((2, _BM, HALF), lambda i: (0, i, 0)),
                  pl.BlockSpec((2, _BM, HALF), lambda i: (0, i, 0)),
                  pl.BlockSpec((2, _BM, 16), lambda i: (0, i, 0)),
                  pl.BlockSpec((2, 1, HALF), lambda i: (0, 0, 0))],
        out_specs=pl.BlockSpec((2, _BM, HALF), lambda i: (0, i, 0)),
        out_shape=jax.ShapeDtypeStruct((2, NP, HALF), jnp.float32),
    )(y, agg, deg_parts, b2)


def _tc3(pooled, cnt, w, b2):
    bm = 512
    return pl.pallas_call(
        _tc3_body,
        grid=(NCP // bm,),
        in_specs=[pl.BlockSpec((2, bm, HALF), lambda i: (0, i, 0)),
                  pl.BlockSpec((bm, 16), lambda i: (i, 0)),
                  pl.BlockSpec((F, F), lambda i: (0, 0)),
                  pl.BlockSpec((1, F), lambda i: (0, 0))],
        out_specs=pl.BlockSpec((bm, F), lambda i: (i, 0)),
        out_shape=jax.ShapeDtypeStruct((NCP, F), jnp.float32),
    )(pooled, cnt, w, b2)


def kernel(x, W_gcn, b_gcn, W_out, b_out, edge_index, cluster_ids):
    src = jnp.concatenate(
        [edge_index[0], jnp.full((EP - E,), N, jnp.int32)])
    dst = jnp.concatenate(
        [edge_index[1], jnp.full((EP - E,), N, jnp.int32)])
    src2 = jnp.concatenate([src, src + NP])
    cl = jnp.concatenate(
        [cluster_ids, jnp.full((NP - N,), NCLUS, jnp.int32)])
    x_pad = jnp.pad(x, ((0, NP - N), (0, 0)))

    degf, cnt = _sc_degree_counts(dst, cl)
    deg_parts = degf.reshape(2, NP, 16)
    y = _tc1(x_pad, W_gcn, deg_parts)
    agg = _sc_edge_agg(y.reshape(2 * NP, HALF), src2, dst)
    h = _tc2(y, agg.reshape(2, NP, HALF), deg_parts, b_gcn.reshape(2, 1, HALF))
    pooled = _sc_pool(h.reshape(2 * NP, HALF), cl)
    outp = _tc3(pooled.reshape(2, NCP, HALF), cnt, W_out, b_out.reshape(1, F))
    return outp[:NCLUS]


# speedup vs baseline: 9.9905x; 1.0308x over previous
import functools

import jax
import jax.numpy as jnp
from jax import lax
from jax.experimental import pallas as pl
from jax.experimental.pallas import tpu as pltpu
from jax.experimental.pallas import tpu_sc as plsc

N = 10000
F = 256
E = 320000
NCLUS = 5000
HALF = 128
CH = 128

NP = 10240
EP = 327680
NCP = 5120

_MESH = plsc.VectorSubcoreMesh(core_axis_name="c", subcore_axis_name="s")


def _zero_rows(ref, nrows, ncols):
    z = jnp.zeros((16,), jnp.float32)

    def body(r, carry):
        for k in range(ncols // 16):
            ref[r, pl.ds(k * 16, 16)] = z
        return carry

    lax.fori_loop(0, nrows, body, 0)


def _fill_onehot(ref, nrows):
    iota = lax.iota(jnp.int32, 16)
    oh = jnp.where(iota == 0, 1.0, 0.0).astype(jnp.float32)
    z = jnp.zeros((16,), jnp.float32)

    def body(r, carry):
        ref[r, pl.ds(0, 16)] = oh
        for k in range(1, HALF // 16):
            ref[r, pl.ds(k * 16, 16)] = z
        return carry

    lax.fori_loop(0, nrows, body, 0)


@functools.partial(
    pl.kernel,
    out_type=jax.ShapeDtypeStruct((2 * NP, HALF), jnp.float32),
    mesh=_MESH,
    scratch_types=[
        pltpu.VMEM((CH,), jnp.int32),
        pltpu.VMEM((CH, HALF), jnp.float32),
        pltpu.VMEM((CH, HALF), jnp.float32),
        pltpu.VMEM_SHARED((NP, HALF), jnp.float32),
    ],
)
def _sc_degree(dst_hbm, deg_out, idx_b, one_b, zero_b, deg_acc):
    c = lax.axis_index("c")
    s = lax.axis_index("s")
    _fill_onehot(one_b, CH)
    _zero_rows(zero_b, CH, HALF)
    for k in range(5):
        pltpu.sync_copy(zero_b, deg_acc.at[pl.ds(s * 640 + k * CH, CH)])
    plsc.subcore_barrier()

    def dbody(j, carry):
        base = ((c * 16 + s) * 80 + j) * CH
        pltpu.sync_copy(dst_hbm.at[pl.ds(base, CH)], idx_b)
        pltpu.sync_copy(one_b, deg_acc.at[idx_b], add=True)
        return carry

    lax.fori_loop(0, 80, dbody, 0)
    plsc.subcore_barrier()
    pltpu.sync_copy(deg_acc.at[pl.ds(s * 640, 640)],
                    deg_out.at[pl.ds(c * NP + s * 640, 640)])


_NBUF = 5
_ECH = 64
_ECHUNKS = 20480 // _ECH


@functools.partial(
    pl.kernel,
    out_type=jax.ShapeDtypeStruct((2 * NP, HALF), jnp.float32),
    mesh=_MESH,
    scratch_types=(
        [pltpu.VMEM((_ECH,), jnp.int32) for _ in range(_NBUF)]
        + [pltpu.VMEM((_ECH,), jnp.int32) for _ in range(_NBUF)]
        + [pltpu.VMEM((_ECH, HALF), jnp.float32) for _ in range(_NBUF)]
        + [pltpu.VMEM_SHARED((NP, HALF), jnp.float32)]
        + [pltpu.SemaphoreType.DMA for _ in range(2 * _NBUF)]
    ),
)
def _sc_edge_agg(y_hbm, src2_hbm, dst_hbm, out_hbm, *refs):
    srcs = refs[0:_NBUF]
    dsts = refs[_NBUF:2 * _NBUF]
    rows = refs[2 * _NBUF:3 * _NBUF]
    acc = refs[3 * _NBUF]
    gsem = refs[3 * _NBUF + 1:3 * _NBUF + 1 + _NBUF]
    ssem = refs[3 * _NBUF + 1 + _NBUF:]
    c = lax.axis_index("c")
    s = lax.axis_index("s")
    _zero_rows(rows[0], _ECH, HALF)
    for k in range(640 // _ECH):
        pltpu.sync_copy(rows[0], acc.at[pl.ds(s * 640 + k * _ECH, _ECH)])
    plsc.subcore_barrier()

    def load_idx(j, p):
        base = (s * _ECHUNKS + j) * _ECH
        pltpu.sync_copy(src2_hbm.at[pl.ds(c * EP + base, _ECH)], srcs[p])
        pltpu.sync_copy(dst_hbm.at[pl.ds(base, _ECH)], dsts[p])

    load_idx(0, 0)
    pltpu.async_copy(y_hbm.at[srcs[0]], rows[0], gsem[0])
    load_idx(1, 1)
    pltpu.async_copy(y_hbm.at[srcs[1]], rows[1], gsem[1])

    def ebody(i, carry):
        for p in range(_NBUF):
            j = _NBUF * i + p
            n = (p + 2) % _NBUF
            pltpu.make_async_copy(y_hbm.at[srcs[p]], rows[p], gsem[p]).wait()

            @pl.when(j < _ECHUNKS - 2)
            def _():
                @pl.when(j >= _NBUF - 2)
                def _():
                    pltpu.make_async_copy(rows[n], acc.at[dsts[n]],
                                          ssem[n]).wait()
                load_idx(j + 2, n)
                pltpu.async_copy(y_hbm.at[srcs[n]], rows[n], gsem[n])

            pltpu.async_copy(rows[p], acc.at[dsts[p]], ssem[p], add=True)
        return carry

    lax.fori_loop(0, _ECHUNKS // _NBUF, ebody, 0)
    for p in range(_NBUF):
        pltpu.make_async_copy(rows[p], acc.at[dsts[p]], ssem[p]).wait()
    plsc.subcore_barrier()
    pltpu.sync_copy(acc.at[pl.ds(s * 640, 640)],
                    out_hbm.at[pl.ds(c * NP + s * 640, 640)])


@functools.partial(
    pl.kernel,
    out_type=(jax.ShapeDtypeStruct((2 * NCP, HALF), jnp.float32),
              jax.ShapeDtypeStruct((NCP, HALF), jnp.float32)),
    mesh=_MESH,
    scratch_types=[
        pltpu.VMEM((CH,), jnp.int32),
        pltpu.VMEM((CH, HALF), jnp.float32),
        pltpu.VMEM((CH, HALF), jnp.float32),
        pltpu.VMEM_SHARED((NCP, HALF), jnp.float32),
        pltpu.VMEM_SHARED((NCP, HALF), jnp.float32),
    ],
)
def _sc_pool(h_hbm, cl_hbm, out_hbm, cnt_out, idx_b, rows_b, one_b, acc, cnt_acc):
    c = lax.axis_index("c")
    s = lax.axis_index("s")
    _zero_rows(rows_b, CH, HALF)
    _fill_onehot(one_b, CH)
    for k in range(2):
        pltpu.sync_copy(rows_b, acc.at[pl.ds(s * 320 + k * CH, CH)])
        pltpu.sync_copy(rows_b, cnt_acc.at[pl.ds(s * 320 + k * CH, CH)])
    pltpu.sync_copy(rows_b.at[pl.ds(0, 64)], acc.at[pl.ds(s * 320 + 256, 64)])
    pltpu.sync_copy(rows_b.at[pl.ds(0, 64)], cnt_acc.at[pl.ds(s * 320 + 256, 64)])
    plsc.subcore_barrier()

    def pbody(j, carry):
        nbase = s * 640 + j * CH
        pltpu.sync_copy(cl_hbm.at[pl.ds(nbase, CH)], idx_b)
        pltpu.sync_copy(h_hbm.at[pl.ds(c * NP + nbase, CH)], rows_b)
        pltpu.sync_copy(rows_b, acc.at[idx_b], add=True)

        @pl.when(c == 0)
        def _():
            pltpu.sync_copy(one_b, cnt_acc.at[idx_b], add=True)
        return carry

    lax.fori_loop(0, 5, pbody, 0)
    plsc.subcore_barrier()
    pltpu.sync_copy(acc.at[pl.ds(s * 320, 320)],
                    out_hbm.at[pl.ds(c * NCP + s * 320, 320)])

    @pl.when(c == 0)
    def _():
        pltpu.sync_copy(cnt_acc.at[pl.ds(s * 320, 320)],
                        cnt_out.at[pl.ds(s * 320, 320)])


def _tc1_body(x_ref, w_ref, degp_ref, y_ref):
    xw = jnp.dot(x_ref[...], w_ref[...], preferred_element_type=jnp.float32)
    d = degp_ref[...]
    y = xw * lax.rsqrt(d[0, :, 0:1] + d[1, :, 0:1] + 1.0)
    y_ref[0, :, :] = y[:, :HALF]
    y_ref[1, :, :] = y[:, HALF:]


def _tc2_body(y_ref, a_ref, degp_ref, b_ref, h_ref):
    d = degp_ref[...]
    dinv = lax.rsqrt(d[0, :, 0:1] + d[1, :, 0:1] + 1.0)
    for half in range(2):
        h_ref[half, :, :] = jnp.tanh(
            (y_ref[half, :, :] + a_ref[half, :, :]) * dinv + b_ref[half, :, :])


def _tc3_body(p_ref, cnt_ref, w_ref, b_ref, o_ref):
    inv = 1.0 / jnp.maximum(cnt_ref[:, 0:1], 1.0)
    p0 = p_ref[0, :, :] * inv
    p1 = p_ref[1, :, :] * inv
    o_ref[...] = (jnp.dot(p0, w_ref[:HALF, :], preferred_element_type=jnp.float32)
                  + jnp.dot(p1, w_ref[HALF:, :], preferred_element_type=jnp.float32)
                  + b_ref[...])


_BM = 256


def _tc1(x_pad, w, deg_parts):
    return pl.pallas_call(
        _tc1_body,
        grid=(NP // _BM,),
        in_specs=[pl.BlockSpec((_BM, F), lambda i: (i, 0)),
                  pl.BlockSpec((F, F), lambda i: (0, 0)),
                  pl.BlockSpec((2, _BM, HALF), lambda i: (0, i, 0))],
        out_specs=pl.BlockSpec((2, _BM, HALF), lambda i: (0, i, 0)),
        out_shape=jax.ShapeDtypeStruct((2, NP, HALF), jnp.float32),
    )(x_pad, w, deg_parts)


def _tc2(y, agg, deg_parts, b2):
    return pl.pallas_call(
        _tc2_body,
        grid=(NP // _BM,),
        in_specs=[pl.BlockSpec((2, _BM, HALF), lambda i: (0, i, 0)),
                  pl.BlockSpec((2, _BM, HALF), lambda i: (0, i, 0)),
                  pl.BlockSpec((2, _BM, HALF), lambda i: (0, i, 0)),
                  pl.BlockSpec((2, 1, HALF), lambda i: (0, 0, 0))],
        out_specs=pl.BlockSpec((2, _BM, HALF), lambda i: (0, i, 0)),
        out_shape=jax.ShapeDtypeStruct((2, NP, HALF), jnp.float32),
    )(y, agg, deg_parts, b2)


def _tc3(pooled, cnt, w, b2):
    bm = 512
    return pl.pallas_call(
        _tc3_body,
        grid=(NCP // bm,),
        in_specs=[pl.BlockSpec((2, bm, HALF), lambda i: (0, i, 0)),
                  pl.BlockSpec((bm, HALF), lambda i: (i, 0)),
                  pl.BlockSpec((F, F), lambda i: (0, 0)),
                  pl.BlockSpec((1, F), lambda i: (0, 0))],
        out_specs=pl.BlockSpec((bm, F), lambda i: (i, 0)),
        out_shape=jax.ShapeDtypeStruct((NCP, F), jnp.float32),
    )(pooled, cnt, w, b2)


def kernel(x, W_gcn, b_gcn, W_out, b_out, edge_index, cluster_ids):
    src = jnp.concatenate(
        [edge_index[0], jnp.full((EP - E,), N, jnp.int32)])
    dst = jnp.concatenate(
        [edge_index[1], jnp.full((EP - E,), N, jnp.int32)])
    src2 = jnp.concatenate([src, src + NP])
    cl = jnp.concatenate(
        [cluster_ids, jnp.full((NP - N,), NCLUS, jnp.int32)])
    x_pad = jnp.pad(x, ((0, NP - N), (0, 0)))

    degf = _sc_degree(dst)
    deg_parts = degf.reshape(2, NP, HALF)
    y = _tc1(x_pad, W_gcn, deg_parts)
    agg = _sc_edge_agg(y.reshape(2 * NP, HALF), src2, dst)
    h = _tc2(y, agg.reshape(2, NP, HALF), deg_parts, b_gcn.reshape(2, 1, HALF))
    pooled, cnt = _sc_pool(h.reshape(2 * NP, HALF), cl)
    outp = _tc3(pooled.reshape(2, NCP, HALF), cnt, W_out, b_out.reshape(1, F))
    return outp[:NCLUS]
